# baseline (speedup 1.0000x reference)
import functools
import jax
import jax.numpy as jnp
from jax import lax
from jax.experimental import pallas as pl
from jax.experimental.pallas import tpu as pltpu
from jax.experimental.pallas import tpu_sc as plsc

CHUNK = 128


def _sc_gather_impl(table, idx):
    info = plsc.get_sparse_core_info()
    nc, ns = info.num_cores, info.num_subcores
    nw = nc * ns
    v, d = table.shape
    b = idx.shape[0]
    assert d % 128 == 0 and b % (8 * nw) == 0, (v, d, b)
    b_per_w = b // nw
    ch = min(512 * 128 // d, b_per_w)
    n_ch = b_per_w // ch
    assert ch * n_ch == b_per_w
    mesh = plsc.VectorSubcoreMesh(core_axis_name="c", subcore_axis_name="s")

    @functools.partial(
        pl.kernel, mesh=mesh,
        out_type=jax.ShapeDtypeStruct((b, d), jnp.float32),
        scratch_types=[
            pltpu.VMEM((ch,), jnp.int32),
            pltpu.VMEM((ch, d), jnp.float32),
            pltpu.SemaphoreType.DMA,
        ],
    )
    def k(table_hbm, idx_hbm, out_hbm, idx_v, rows_v, sem):
        wid = lax.axis_index("s") * nc + lax.axis_index("c")
        base = wid * b_per_w
        for c in range(n_ch):
            pltpu.sync_copy(idx_hbm.at[pl.ds(base + c * ch, ch)], idx_v)
            pltpu.async_copy(table_hbm.at[idx_v], rows_v, sem).wait()
            pltpu.sync_copy(rows_v, out_hbm.at[pl.ds(base + c * ch, ch)])

    return k(table, idx)


def _sc_gather(table, idx, width):
    d0 = table.shape[1]
    dp = -(-d0 // 128) * 128
    if dp != d0:
        table = jnp.pad(table, ((0, 0), (0, dp - d0)))
    return _sc_gather_impl(table, idx)[:, :width]


def _lrelu(v, s):
    return jnp.where(v >= 0, v, s * v)


def _head_kernel(h_ref, w1_ref, b1_ref, w2_ref, b2_ref, w3_ref, b3_ref, o_ref):
    h = h_ref[...]
    a = jnp.maximum(h @ w1_ref[...].T + b1_ref[...], 0.0)
    b = a @ w2_ref[...].T + b2_ref[...]
    o_ref[...] = b @ w3_ref[...].T + b3_ref[...]


def _head(h, params):
    n = h.shape[0]
    blk = 4096
    w1, b1 = params["h1_W"], params["h1_b"]
    w2, b2 = params["h2_W"], params["h2_b"]
    w3, b3 = params["lin_W"], params["lin_b"]
    out = pl.pallas_call(
        _head_kernel,
        grid=(n // blk,),
        in_specs=[
            pl.BlockSpec((blk, h.shape[1]), lambda i: (i, 0)),
            pl.BlockSpec(w1.shape, lambda i: (0, 0)),
            pl.BlockSpec(b1.shape, lambda i: (0,)),
            pl.BlockSpec(w2.shape, lambda i: (0, 0)),
            pl.BlockSpec(b2.shape, lambda i: (0,)),
            pl.BlockSpec(w3.shape, lambda i: (0, 0)),
            pl.BlockSpec(b3.shape, lambda i: (0,)),
        ],
        out_specs=pl.BlockSpec((blk, w3.shape[0]), lambda i: (i, 0)),
        out_shape=jax.ShapeDtypeStruct((n, w3.shape[0]), h.dtype),
    )(h, w1, b1, w2, b2, w3, b3)
    return out


_I32_MAX = jnp.iinfo(jnp.int32).max


def _aug_q(p):
    n2 = jnp.sum(p * p, axis=1, keepdims=True)
    one = jnp.ones_like(n2)
    zero = jnp.zeros((p.shape[0], 3), p.dtype)
    return jnp.concatenate([p, n2, one, zero], axis=1)


def _aug_s(p):
    n2 = jnp.sum(p * p, axis=1, keepdims=True)
    one = jnp.ones_like(n2)
    zero = jnp.zeros((p.shape[0], 3), p.dtype)
    return jnp.concatenate([-2.0 * p, one, n2, zero], axis=1)


_KNN_STRIP = 512


def _knn16_body(q_ref, s_ref, o_ref):
    tq = q_ref.shape[0]
    s_cols = s_ref.shape[0]
    strip = min(_KNN_STRIP, s_cols)
    q = q_ref[...]
    ms = tuple(jnp.full((tq, 128), _I32_MAX, jnp.int32) for _ in range(4))
    for sb in range(s_cols // strip):
        d = jax.lax.dot_general(
            q, s_ref[sb * strip:(sb + 1) * strip, :],
            (((1,), (1,)), ((), ())),
            preferred_element_type=jnp.float32,
            precision=jax.lax.Precision.HIGHEST)
        for jj in range(strip // 128):
            j = sb * (strip // 128) + jj
            m1, m2, m3, m4 = ms
            x = jnp.maximum(d[:, jj * 128:(jj + 1) * 128], 0.0)
            x = jax.lax.bitcast_convert_type(x, jnp.int32)
            x = (x & ~0xFF) | j
            t = jnp.minimum(m1, x); x = jnp.maximum(m1, x); m1 = t
            t = jnp.minimum(m2, x); x = jnp.maximum(m2, x); m2 = t
            t = jnp.minimum(m3, x); x = jnp.maximum(m3, x); m3 = t
            m4 = jnp.minimum(m4, x)
            ms = (m1, m2, m3, m4)

    cand = jnp.concatenate(ms, axis=1)
    iota = jax.lax.broadcasted_iota(jnp.int32, (tq, 512), 1)
    for kk in range(16):
        mn = jnp.min(cand, axis=1, keepdims=True)
        pos = jnp.min(jnp.where(cand == mn, iota, _I32_MAX), axis=1, keepdims=True)
        col = ((mn & 0xFF) << 7) | (pos & 127)
        o_ref[:, kk:kk + 1] = col
        cand = jnp.where(iota == pos, _I32_MAX, cand)


def _knn_idx(q, s, k, tq=None):
    assert k == 16
    qn, sn = q.shape[0], s.shape[0]
    if tq is None:
        tq = min(qn, 64)
    qa, sa = _aug_q(q), _aug_s(s)
    out = pl.pallas_call(
        _knn16_body,
        grid=(qn // tq,),
        in_specs=[
            pl.BlockSpec((tq, 8), lambda i: (i, 0)),
            pl.BlockSpec((sn, 8), lambda i: (0, 0)),
        ],
        out_specs=pl.BlockSpec((tq, 16), lambda i: (i, 0)),
        out_shape=jax.ShapeDtypeStruct((qn, 16), jnp.int32),
    )(qa, sa)
    return out


def _nn1_body(q_ref, s_ref, o_ref):
    tq = q_ref.shape[0]
    s_cols = s_ref.shape[0]
    strip = min(_KNN_STRIP, s_cols)
    q = q_ref[...]
    mv = jnp.full((tq, 128), jnp.inf, jnp.float32)
    mi = jnp.zeros((tq, 128), jnp.int32)
    for sb in range(s_cols // strip):
        d = jax.lax.dot_general(
            q, s_ref[sb * strip:(sb + 1) * strip, :],
            (((1,), (1,)), ((), ())),
            preferred_element_type=jnp.float32,
            precision=jax.lax.Precision.HIGHEST)
        for jj in range(strip // 128):
            j = sb * (strip // 128) + jj
            x = d[:, jj * 128:(jj + 1) * 128]
            upd = x < mv
            mv = jnp.where(upd, x, mv)
            mi = jnp.where(upd, j, mi)

    v = jnp.min(mv, axis=1, keepdims=True)
    lane = jax.lax.broadcasted_iota(jnp.int32, (tq, 128), 1)
    colf = (mi << 7) | lane
    col = jnp.min(jnp.where(mv == v, colf, _I32_MAX), axis=1, keepdims=True)
    o_ref[...] = col


def _nn1(q, s, tq=None):
    qn, sn = q.shape[0], s.shape[0]
    if tq is None:
        tq = min(qn, 128)
    qa, sa = _aug_q(q), _aug_s(s)
    out = pl.pallas_call(
        _nn1_body,
        grid=(qn // tq,),
        in_specs=[
            pl.BlockSpec((tq, 8), lambda i: (i, 0)),
            pl.BlockSpec((sn, 8), lambda i: (0, 0)),
        ],
        out_specs=pl.BlockSpec((tq, 1), lambda i: (i, 0)),
        out_shape=jax.ShapeDtypeStruct((qn, 1), jnp.int32),
    )(qa, sa)
    return out.reshape(qn)


def _lfa_edges(params, pfx, x_j, pos_i, pos_j, n_q, k):
    dist = pos_j - pos_i
    eu = jnp.sum(jnp.abs(dist), axis=1, keepdims=True)
    rel = jnp.concatenate([pos_i, pos_j, dist, eu], axis=1)
    lse = rel @ params[pfx + "e_W"].T + params[pfx + "e_b"]
    out1 = jnp.concatenate([x_j, lse], axis=1)
    att = jax.nn.softmax(out1 @ params[pfx + "a_W"].T + params[pfx + "a_b"], axis=-1)
    msg = att * out1
    return msg.reshape(n_q, k, msg.shape[1]).sum(axis=1)


def _block(params, pfx, x, pos, decimation, k):
    n = x.shape[0]
    idx = jnp.arange(0, n, decimation)
    n_q = idx.shape[0]
    q_pos = pos[idx]
    nbrs = _knn_idx(q_pos, pos, k)
    col = nbrs.reshape(-1)

    h0 = _lrelu(x @ params[pfx + "_m1_W"].T + params[pfx + "_m1_b"], 0.2)
    c1 = h0.shape[1]
    g1 = _sc_gather(jnp.concatenate([h0, pos], axis=1), col, c1 + 3)
    x_j1, pos_j = g1[:, :c1], g1[:, c1:]
    pos_i = jnp.repeat(pos[:n_q], k, axis=0)
    h1 = _lfa_edges(params, pfx + "_l1_", x_j1, pos_i, pos_j, n_q, k)
    x_j2 = _sc_gather(h1, jnp.minimum(col, n_q - 1), h1.shape[1])
    x_j2 = jnp.where((col < n_q)[:, None], x_j2, 0.0)
    h2 = _lfa_edges(params, pfx + "_l2_", x_j2, pos_i, pos_j, n_q, k)
    h2_idx = jnp.where((idx < n_q)[:, None], h2[jnp.minimum(idx, n_q - 1)], 0.0)
    m2 = _lrelu(h2_idx @ params[pfx + "_m2_W"].T + params[pfx + "_m2_b"], 0.2)
    sc = _lrelu(x[idx] @ params[pfx + "_sc_W"].T + params[pfx + "_sc_b"], 0.2)
    out = _lrelu(m2 + sc, 0.01)
    return out, q_pos


def _fp(params, pfx, xh, pos, pos_skip, x_skip):
    nn = _nn1(pos_skip, pos)
    xi = _sc_gather(xh, nn, xh.shape[1])
    if x_skip is not None:
        xi = jnp.concatenate([xi, x_skip], axis=1)
    return xi @ params[pfx + "_W"].T + params[pfx + "_b"]


@jax.jit
def _forward(x, pos, params):
    x0, p0 = x, pos
    x1, p1 = _block(params, "b1", x0, p0, 4, 16)
    x2, p2 = _block(params, "b2", x1, p1, 4, 16)
    x3, p3 = _block(params, "b3", x2, p2, 4, 16)
    x4, p4 = _block(params, "b4", x3, p3, 4, 16)
    h = x4 @ params["mlp1_W"].T + params["mlp1_b"]
    h = _fp(params, "fp4", h, p4, p3, x3)
    h = _fp(params, "fp3", h, p3, p2, x2)
    h = _fp(params, "fp2", h, p2, p1, x1)
    h = _fp(params, "fp1", h, p1, p0, x0)
    return _head(h, params)


def kernel(x, pos, batch, params):
    return _forward(x, pos, params)


# speedup vs baseline: 1.2687x; 1.2687x over previous
import functools
import jax
import jax.numpy as jnp
from jax import lax
from jax.experimental import pallas as pl
from jax.experimental.pallas import tpu as pltpu
from jax.experimental.pallas import tpu_sc as plsc

CHUNK = 128


def _sc_gather_impl(table, idx):
    info = plsc.get_sparse_core_info()
    nc, ns = info.num_cores, info.num_subcores
    nw = nc * ns
    v, d = table.shape
    b = idx.shape[0]
    assert d % 128 == 0 and b % (8 * nw) == 0, (v, d, b)
    b_per_w = b // nw
    ch = min(512 * 128 // d, b_per_w)
    n_ch = b_per_w // ch
    assert ch * n_ch == b_per_w
    mesh = plsc.VectorSubcoreMesh(core_axis_name="c", subcore_axis_name="s")

    @functools.partial(
        pl.kernel, mesh=mesh,
        out_type=jax.ShapeDtypeStruct((b, d), jnp.float32),
        scratch_types=[
            pltpu.VMEM((ch,), jnp.int32),
            pltpu.VMEM((ch, d), jnp.float32),
            pltpu.SemaphoreType.DMA,
        ],
    )
    def k(table_hbm, idx_hbm, out_hbm, idx_v, rows_v, sem):
        wid = lax.axis_index("s") * nc + lax.axis_index("c")
        base = wid * b_per_w
        for c in range(n_ch):
            pltpu.sync_copy(idx_hbm.at[pl.ds(base + c * ch, ch)], idx_v)
            pltpu.async_copy(table_hbm.at[idx_v], rows_v, sem).wait()
            pltpu.sync_copy(rows_v, out_hbm.at[pl.ds(base + c * ch, ch)])

    return k(table, idx)


def _sc_gather(table, idx, width):
    if idx.shape[0] < 131072:
        return table[idx][:, :width]
    d0 = table.shape[1]
    dp = -(-d0 // 128) * 128
    if dp != d0:
        table = jnp.pad(table, ((0, 0), (0, dp - d0)))
    return _sc_gather_impl(table, idx)[:, :width]


def _lrelu(v, s):
    return jnp.where(v >= 0, v, s * v)


def _head_kernel(h_ref, w1_ref, b1_ref, w2_ref, b2_ref, w3_ref, b3_ref, o_ref):
    h = h_ref[...]
    a = jnp.maximum(h @ w1_ref[...].T + b1_ref[...], 0.0)
    b = a @ w2_ref[...].T + b2_ref[...]
    o_ref[...] = b @ w3_ref[...].T + b3_ref[...]


def _head(h, params):
    n = h.shape[0]
    blk = 4096
    w1, b1 = params["h1_W"], params["h1_b"]
    w2, b2 = params["h2_W"], params["h2_b"]
    w3, b3 = params["lin_W"], params["lin_b"]
    out = pl.pallas_call(
        _head_kernel,
        grid=(n // blk,),
        in_specs=[
            pl.BlockSpec((blk, h.shape[1]), lambda i: (i, 0)),
            pl.BlockSpec(w1.shape, lambda i: (0, 0)),
            pl.BlockSpec(b1.shape, lambda i: (0,)),
            pl.BlockSpec(w2.shape, lambda i: (0, 0)),
            pl.BlockSpec(b2.shape, lambda i: (0,)),
            pl.BlockSpec(w3.shape, lambda i: (0, 0)),
            pl.BlockSpec(b3.shape, lambda i: (0,)),
        ],
        out_specs=pl.BlockSpec((blk, w3.shape[0]), lambda i: (i, 0)),
        out_shape=jax.ShapeDtypeStruct((n, w3.shape[0]), h.dtype),
    )(h, w1, b1, w2, b2, w3, b3)
    return out


_I32_MAX = jnp.iinfo(jnp.int32).max


def _aug_q(p):
    n2 = jnp.sum(p * p, axis=1, keepdims=True)
    one = jnp.ones_like(n2)
    zero = jnp.zeros((p.shape[0], 3), p.dtype)
    return jnp.concatenate([p, n2, one, zero], axis=1)


def _aug_s(p):
    n2 = jnp.sum(p * p, axis=1, keepdims=True)
    one = jnp.ones_like(n2)
    zero = jnp.zeros((p.shape[0], 3), p.dtype)
    return jnp.concatenate([-2.0 * p, one, n2, zero], axis=1)


_KNN_STRIP = 512


def _knn16_body(q_ref, s_ref, o_ref):
    tq = q_ref.shape[0]
    s_cols = s_ref.shape[0]
    strip = min(_KNN_STRIP, s_cols)
    q = q_ref[...]
    ms = tuple(jnp.full((tq, 128), _I32_MAX, jnp.int32) for _ in range(4))
    for sb in range(s_cols // strip):
        d = jax.lax.dot_general(
            q, s_ref[sb * strip:(sb + 1) * strip, :],
            (((1,), (1,)), ((), ())),
            preferred_element_type=jnp.float32,
            precision=jax.lax.Precision.HIGHEST)
        for jj in range(strip // 128):
            j = sb * (strip // 128) + jj
            m1, m2, m3, m4 = ms
            x = jnp.maximum(d[:, jj * 128:(jj + 1) * 128], 0.0)
            x = jax.lax.bitcast_convert_type(x, jnp.int32)
            x = (x & ~0xFF) | j
            t = jnp.minimum(m1, x); x = jnp.maximum(m1, x); m1 = t
            t = jnp.minimum(m2, x); x = jnp.maximum(m2, x); m2 = t
            t = jnp.minimum(m3, x); x = jnp.maximum(m3, x); m3 = t
            m4 = jnp.minimum(m4, x)
            ms = (m1, m2, m3, m4)

    cand = jnp.concatenate(ms, axis=1)
    iota = jax.lax.broadcasted_iota(jnp.int32, (tq, 512), 1)
    for kk in range(16):
        mn = jnp.min(cand, axis=1, keepdims=True)
        pos = jnp.min(jnp.where(cand == mn, iota, _I32_MAX), axis=1, keepdims=True)
        col = ((mn & 0xFF) << 7) | (pos & 127)
        o_ref[:, kk:kk + 1] = col
        cand = jnp.where(iota == pos, _I32_MAX, cand)


def _knn_idx(q, s, k, tq=None):
    assert k == 16
    qn, sn = q.shape[0], s.shape[0]
    if tq is None:
        tq = min(qn, 64)
    qa, sa = _aug_q(q), _aug_s(s)
    out = pl.pallas_call(
        _knn16_body,
        grid=(qn // tq,),
        in_specs=[
            pl.BlockSpec((tq, 8), lambda i: (i, 0)),
            pl.BlockSpec((sn, 8), lambda i: (0, 0)),
        ],
        out_specs=pl.BlockSpec((tq, 16), lambda i: (i, 0)),
        out_shape=jax.ShapeDtypeStruct((qn, 16), jnp.int32),
    )(qa, sa)
    return out


def _nn1_body(q_ref, s_ref, o_ref):
    tq = q_ref.shape[0]
    s_cols = s_ref.shape[0]
    strip = min(_KNN_STRIP, s_cols)
    q = q_ref[...]
    mv = jnp.full((tq, 128), jnp.inf, jnp.float32)
    mi = jnp.zeros((tq, 128), jnp.int32)
    for sb in range(s_cols // strip):
        d = jax.lax.dot_general(
            q, s_ref[sb * strip:(sb + 1) * strip, :],
            (((1,), (1,)), ((), ())),
            preferred_element_type=jnp.float32,
            precision=jax.lax.Precision.HIGHEST)
        for jj in range(strip // 128):
            j = sb * (strip // 128) + jj
            x = d[:, jj * 128:(jj + 1) * 128]
            upd = x < mv
            mv = jnp.where(upd, x, mv)
            mi = jnp.where(upd, j, mi)

    v = jnp.min(mv, axis=1, keepdims=True)
    lane = jax.lax.broadcasted_iota(jnp.int32, (tq, 128), 1)
    colf = (mi << 7) | lane
    col = jnp.min(jnp.where(mv == v, colf, _I32_MAX), axis=1, keepdims=True)
    o_ref[...] = col


def _nn1(q, s, tq=None):
    qn, sn = q.shape[0], s.shape[0]
    if tq is None:
        tq = min(qn, 128)
    qa, sa = _aug_q(q), _aug_s(s)
    out = pl.pallas_call(
        _nn1_body,
        grid=(qn // tq,),
        in_specs=[
            pl.BlockSpec((tq, 8), lambda i: (i, 0)),
            pl.BlockSpec((sn, 8), lambda i: (0, 0)),
        ],
        out_specs=pl.BlockSpec((tq, 1), lambda i: (i, 0)),
        out_shape=jax.ShapeDtypeStruct((qn, 1), jnp.int32),
    )(qa, sa)
    return out.reshape(qn)


def _lfa_edges(params, pfx, x_j, pos_i, pos_j, n_q, k):
    dist = pos_j - pos_i
    eu = jnp.sum(jnp.abs(dist), axis=1, keepdims=True)
    rel = jnp.concatenate([pos_i, pos_j, dist, eu], axis=1)
    lse = rel @ params[pfx + "e_W"].T + params[pfx + "e_b"]
    out1 = jnp.concatenate([x_j, lse], axis=1)
    att = jax.nn.softmax(out1 @ params[pfx + "a_W"].T + params[pfx + "a_b"], axis=-1)
    msg = att * out1
    return msg.reshape(n_q, k, msg.shape[1]).sum(axis=1)


def _block(params, pfx, x, pos, decimation, k):
    n = x.shape[0]
    idx = jnp.arange(0, n, decimation)
    n_q = idx.shape[0]
    q_pos = pos[idx]
    nbrs = _knn_idx(q_pos, pos, k)
    col = nbrs.reshape(-1)

    h0 = _lrelu(x @ params[pfx + "_m1_W"].T + params[pfx + "_m1_b"], 0.2)
    c1 = h0.shape[1]
    g1 = _sc_gather(jnp.concatenate([h0, pos], axis=1), col, c1 + 3)
    x_j1, pos_j = g1[:, :c1], g1[:, c1:]
    pos_i = jnp.repeat(pos[:n_q], k, axis=0)
    h1 = _lfa_edges(params, pfx + "_l1_", x_j1, pos_i, pos_j, n_q, k)
    x_j2 = _sc_gather(h1, jnp.minimum(col, n_q - 1), h1.shape[1])
    x_j2 = jnp.where((col < n_q)[:, None], x_j2, 0.0)
    h2 = _lfa_edges(params, pfx + "_l2_", x_j2, pos_i, pos_j, n_q, k)
    h2_idx = jnp.where((idx < n_q)[:, None], h2[jnp.minimum(idx, n_q - 1)], 0.0)
    m2 = _lrelu(h2_idx @ params[pfx + "_m2_W"].T + params[pfx + "_m2_b"], 0.2)
    sc = _lrelu(x[idx] @ params[pfx + "_sc_W"].T + params[pfx + "_sc_b"], 0.2)
    out = _lrelu(m2 + sc, 0.01)
    return out, q_pos


def _fp(params, pfx, xh, pos, pos_skip, x_skip):
    nn = _nn1(pos_skip, pos)
    xi = _sc_gather(xh, nn, xh.shape[1])
    if x_skip is not None:
        xi = jnp.concatenate([xi, x_skip], axis=1)
    return xi @ params[pfx + "_W"].T + params[pfx + "_b"]


@jax.jit
def _forward(x, pos, params):
    x0, p0 = x, pos
    x1, p1 = _block(params, "b1", x0, p0, 4, 16)
    x2, p2 = _block(params, "b2", x1, p1, 4, 16)
    x3, p3 = _block(params, "b3", x2, p2, 4, 16)
    x4, p4 = _block(params, "b4", x3, p3, 4, 16)
    h = x4 @ params["mlp1_W"].T + params["mlp1_b"]
    h = _fp(params, "fp4", h, p4, p3, x3)
    h = _fp(params, "fp3", h, p3, p2, x2)
    h = _fp(params, "fp2", h, p2, p1, x1)
    h = _fp(params, "fp1", h, p1, p0, x0)
    return _head(h, params)


def kernel(x, pos, batch, params):
    return _forward(x, pos, params)


# speedup vs baseline: 1.5590x; 1.2288x over previous
import functools
import jax
import jax.numpy as jnp
from jax import lax
from jax.experimental import pallas as pl
from jax.experimental.pallas import tpu as pltpu
from jax.experimental.pallas import tpu_sc as plsc

CHUNK = 128


def _sc_gather_impl(table, idx):
    info = plsc.get_sparse_core_info()
    nc, ns = info.num_cores, info.num_subcores
    nw = nc * ns
    v, d = table.shape
    b = idx.shape[0]
    assert d % 128 == 0 and b % (8 * nw) == 0, (v, d, b)
    b_per_w = b // nw
    ch = min(512 * 128 // d, b_per_w)
    n_ch = b_per_w // ch
    assert ch * n_ch == b_per_w
    mesh = plsc.VectorSubcoreMesh(core_axis_name="c", subcore_axis_name="s")

    @functools.partial(
        pl.kernel, mesh=mesh,
        out_type=jax.ShapeDtypeStruct((b, d), jnp.float32),
        scratch_types=[
            pltpu.VMEM((ch,), jnp.int32),
            pltpu.VMEM((ch, d), jnp.float32),
            pltpu.SemaphoreType.DMA,
        ],
    )
    def k(table_hbm, idx_hbm, out_hbm, idx_v, rows_v, sem):
        wid = lax.axis_index("s") * nc + lax.axis_index("c")
        base = wid * b_per_w
        for c in range(n_ch):
            pltpu.sync_copy(idx_hbm.at[pl.ds(base + c * ch, ch)], idx_v)
            pltpu.async_copy(table_hbm.at[idx_v], rows_v, sem).wait()
            pltpu.sync_copy(rows_v, out_hbm.at[pl.ds(base + c * ch, ch)])

    return k(table, idx)


def _sc_gather(table, idx, width):
    if idx.shape[0] < 10 ** 9:
        return table[idx][:, :width]
    d0 = table.shape[1]
    dp = -(-d0 // 128) * 128
    if dp != d0:
        table = jnp.pad(table, ((0, 0), (0, dp - d0)))
    return _sc_gather_impl(table, idx)[:, :width]


def _lrelu(v, s):
    return jnp.where(v >= 0, v, s * v)


def _head_kernel(h_ref, w1_ref, b1_ref, w2_ref, b2_ref, w3_ref, b3_ref, o_ref):
    h = h_ref[...]
    a = jnp.maximum(h @ w1_ref[...].T + b1_ref[...], 0.0)
    b = a @ w2_ref[...].T + b2_ref[...]
    o_ref[...] = b @ w3_ref[...].T + b3_ref[...]


def _head(h, params):
    n = h.shape[0]
    blk = 4096
    w1, b1 = params["h1_W"], params["h1_b"]
    w2, b2 = params["h2_W"], params["h2_b"]
    w3, b3 = params["lin_W"], params["lin_b"]
    out = pl.pallas_call(
        _head_kernel,
        grid=(n // blk,),
        in_specs=[
            pl.BlockSpec((blk, h.shape[1]), lambda i: (i, 0)),
            pl.BlockSpec(w1.shape, lambda i: (0, 0)),
            pl.BlockSpec(b1.shape, lambda i: (0,)),
            pl.BlockSpec(w2.shape, lambda i: (0, 0)),
            pl.BlockSpec(b2.shape, lambda i: (0,)),
            pl.BlockSpec(w3.shape, lambda i: (0, 0)),
            pl.BlockSpec(b3.shape, lambda i: (0,)),
        ],
        out_specs=pl.BlockSpec((blk, w3.shape[0]), lambda i: (i, 0)),
        out_shape=jax.ShapeDtypeStruct((n, w3.shape[0]), h.dtype),
    )(h, w1, b1, w2, b2, w3, b3)
    return out


_I32_MAX = jnp.iinfo(jnp.int32).max


def _aug_q(p):
    n2 = jnp.sum(p * p, axis=1, keepdims=True)
    one = jnp.ones_like(n2)
    zero = jnp.zeros((p.shape[0], 3), p.dtype)
    return jnp.concatenate([p, n2, one, zero], axis=1)


def _aug_s(p):
    n2 = jnp.sum(p * p, axis=1, keepdims=True)
    one = jnp.ones_like(n2)
    zero = jnp.zeros((p.shape[0], 3), p.dtype)
    return jnp.concatenate([-2.0 * p, one, n2, zero], axis=1)


_KNN_STRIP = 512


def _knn16_body(q_ref, s_ref, o_ref):
    tq = q_ref.shape[0]
    s_cols = s_ref.shape[0]
    strip = min(_KNN_STRIP, s_cols)
    q = q_ref[...]
    ms = tuple(jnp.full((tq, 128), _I32_MAX, jnp.int32) for _ in range(4))
    for sb in range(s_cols // strip):
        d = jax.lax.dot_general(
            q, s_ref[sb * strip:(sb + 1) * strip, :],
            (((1,), (1,)), ((), ())),
            preferred_element_type=jnp.float32,
            precision=jax.lax.Precision.HIGHEST)
        for jj in range(strip // 128):
            j = sb * (strip // 128) + jj
            m1, m2, m3, m4 = ms
            x = jnp.maximum(d[:, jj * 128:(jj + 1) * 128], 0.0)
            x = jax.lax.bitcast_convert_type(x, jnp.int32)
            x = (x & ~0xFF) | j
            t = jnp.minimum(m1, x); x = jnp.maximum(m1, x); m1 = t
            t = jnp.minimum(m2, x); x = jnp.maximum(m2, x); m2 = t
            t = jnp.minimum(m3, x); x = jnp.maximum(m3, x); m3 = t
            m4 = jnp.minimum(m4, x)
            ms = (m1, m2, m3, m4)

    cand = jnp.concatenate(ms, axis=1)
    iota = jax.lax.broadcasted_iota(jnp.int32, (tq, 512), 1)
    for kk in range(16):
        mn = jnp.min(cand, axis=1, keepdims=True)
        pos = jnp.min(jnp.where(cand == mn, iota, _I32_MAX), axis=1, keepdims=True)
        col = ((mn & 0xFF) << 7) | (pos & 127)
        o_ref[:, kk:kk + 1] = col
        cand = jnp.where(iota == pos, _I32_MAX, cand)


def _knn_idx(q, s, k, tq=None):
    assert k == 16
    qn, sn = q.shape[0], s.shape[0]
    if tq is None:
        tq = min(qn, 64)
    qa, sa = _aug_q(q), _aug_s(s)
    out = pl.pallas_call(
        _knn16_body,
        grid=(qn // tq,),
        in_specs=[
            pl.BlockSpec((tq, 8), lambda i: (i, 0)),
            pl.BlockSpec((sn, 8), lambda i: (0, 0)),
        ],
        out_specs=pl.BlockSpec((tq, 16), lambda i: (i, 0)),
        out_shape=jax.ShapeDtypeStruct((qn, 16), jnp.int32),
    )(qa, sa)
    return out


def _nn1_body(q_ref, s_ref, o_ref):
    tq = q_ref.shape[0]
    s_cols = s_ref.shape[0]
    strip = min(_KNN_STRIP, s_cols)
    q = q_ref[...]
    mv = jnp.full((tq, 128), jnp.inf, jnp.float32)
    mi = jnp.zeros((tq, 128), jnp.int32)
    for sb in range(s_cols // strip):
        d = jax.lax.dot_general(
            q, s_ref[sb * strip:(sb + 1) * strip, :],
            (((1,), (1,)), ((), ())),
            preferred_element_type=jnp.float32,
            precision=jax.lax.Precision.HIGHEST)
        for jj in range(strip // 128):
            j = sb * (strip // 128) + jj
            x = d[:, jj * 128:(jj + 1) * 128]
            upd = x < mv
            mv = jnp.where(upd, x, mv)
            mi = jnp.where(upd, j, mi)

    v = jnp.min(mv, axis=1, keepdims=True)
    lane = jax.lax.broadcasted_iota(jnp.int32, (tq, 128), 1)
    colf = (mi << 7) | lane
    col = jnp.min(jnp.where(mv == v, colf, _I32_MAX), axis=1, keepdims=True)
    o_ref[...] = col


def _nn1(q, s, tq=None):
    qn, sn = q.shape[0], s.shape[0]
    if tq is None:
        tq = min(qn, 128)
    qa, sa = _aug_q(q), _aug_s(s)
    out = pl.pallas_call(
        _nn1_body,
        grid=(qn // tq,),
        in_specs=[
            pl.BlockSpec((tq, 8), lambda i: (i, 0)),
            pl.BlockSpec((sn, 8), lambda i: (0, 0)),
        ],
        out_specs=pl.BlockSpec((tq, 1), lambda i: (i, 0)),
        out_shape=jax.ShapeDtypeStruct((qn, 1), jnp.int32),
    )(qa, sa)
    return out.reshape(qn)


def _lfa_edges(params, pfx, x_j, pos_i, pos_j, n_q, k):
    dist = pos_j - pos_i
    eu = jnp.sum(jnp.abs(dist), axis=1, keepdims=True)
    rel = jnp.concatenate([pos_i, pos_j, dist, eu], axis=1)
    lse = rel @ params[pfx + "e_W"].T + params[pfx + "e_b"]
    out1 = jnp.concatenate([x_j, lse], axis=1)
    att = jax.nn.softmax(out1 @ params[pfx + "a_W"].T + params[pfx + "a_b"], axis=-1)
    msg = att * out1
    return msg.reshape(n_q, k, msg.shape[1]).sum(axis=1)


def _block(params, pfx, x, pos, decimation, k):
    n = x.shape[0]
    idx = jnp.arange(0, n, decimation)
    n_q = idx.shape[0]
    q_pos = pos[idx]
    nbrs = _knn_idx(q_pos, pos, k)
    col = nbrs.reshape(-1)

    h0 = _lrelu(x @ params[pfx + "_m1_W"].T + params[pfx + "_m1_b"], 0.2)
    c1 = h0.shape[1]
    g1 = _sc_gather(jnp.concatenate([h0, pos], axis=1), col, c1 + 3)
    x_j1, pos_j = g1[:, :c1], g1[:, c1:]
    pos_i = jnp.repeat(pos[:n_q], k, axis=0)
    h1 = _lfa_edges(params, pfx + "_l1_", x_j1, pos_i, pos_j, n_q, k)
    x_j2 = _sc_gather(h1, jnp.minimum(col, n_q - 1), h1.shape[1])
    x_j2 = jnp.where((col < n_q)[:, None], x_j2, 0.0)
    h2 = _lfa_edges(params, pfx + "_l2_", x_j2, pos_i, pos_j, n_q, k)
    h2_idx = jnp.where((idx < n_q)[:, None], h2[jnp.minimum(idx, n_q - 1)], 0.0)
    m2 = _lrelu(h2_idx @ params[pfx + "_m2_W"].T + params[pfx + "_m2_b"], 0.2)
    sc = _lrelu(x[idx] @ params[pfx + "_sc_W"].T + params[pfx + "_sc_b"], 0.2)
    out = _lrelu(m2 + sc, 0.01)
    return out, q_pos


def _fp(params, pfx, xh, pos, pos_skip, x_skip):
    nn = _nn1(pos_skip, pos)
    xi = _sc_gather(xh, nn, xh.shape[1])
    if x_skip is not None:
        xi = jnp.concatenate([xi, x_skip], axis=1)
    return xi @ params[pfx + "_W"].T + params[pfx + "_b"]


@jax.jit
def _forward(x, pos, params):
    x0, p0 = x, pos
    x1, p1 = _block(params, "b1", x0, p0, 4, 16)
    x2, p2 = _block(params, "b2", x1, p1, 4, 16)
    x3, p3 = _block(params, "b3", x2, p2, 4, 16)
    x4, p4 = _block(params, "b4", x3, p3, 4, 16)
    h = x4 @ params["mlp1_W"].T + params["mlp1_b"]
    h = _fp(params, "fp4", h, p4, p3, x3)
    h = _fp(params, "fp3", h, p3, p2, x2)
    h = _fp(params, "fp2", h, p2, p1, x1)
    h = _fp(params, "fp1", h, p1, p0, x0)
    return _head(h, params)


def kernel(x, pos, batch, params):
    return _forward(x, pos, params)


# speedup vs baseline: 2.0579x; 1.3200x over previous
import functools
import jax
import jax.numpy as jnp
from jax import lax
from jax.experimental import pallas as pl
from jax.experimental.pallas import tpu as pltpu
from jax.experimental.pallas import tpu_sc as plsc

CHUNK = 128


def _sc_gather_impl(table, idx):
    info = plsc.get_sparse_core_info()
    nc, ns = info.num_cores, info.num_subcores
    nw = nc * ns
    v, d = table.shape
    b = idx.shape[0]
    assert d % 128 == 0 and b % (8 * nw) == 0, (v, d, b)
    b_per_w = b // nw
    ch = min(512 * 128 // d, b_per_w)
    n_ch = b_per_w // ch
    assert ch * n_ch == b_per_w
    mesh = plsc.VectorSubcoreMesh(core_axis_name="c", subcore_axis_name="s")

    @functools.partial(
        pl.kernel, mesh=mesh,
        out_type=jax.ShapeDtypeStruct((b, d), jnp.float32),
        scratch_types=[
            pltpu.VMEM((ch,), jnp.int32),
            pltpu.VMEM((ch, d), jnp.float32),
            pltpu.SemaphoreType.DMA,
        ],
    )
    def k(table_hbm, idx_hbm, out_hbm, idx_v, rows_v, sem):
        wid = lax.axis_index("s") * nc + lax.axis_index("c")
        base = wid * b_per_w
        for c in range(n_ch):
            pltpu.sync_copy(idx_hbm.at[pl.ds(base + c * ch, ch)], idx_v)
            pltpu.async_copy(table_hbm.at[idx_v], rows_v, sem).wait()
            pltpu.sync_copy(rows_v, out_hbm.at[pl.ds(base + c * ch, ch)])

    return k(table, idx)


def _sc_gather(table, idx, width):
    if idx.shape[0] < 10 ** 9:
        return table[idx][:, :width]
    d0 = table.shape[1]
    dp = -(-d0 // 128) * 128
    if dp != d0:
        table = jnp.pad(table, ((0, 0), (0, dp - d0)))
    return _sc_gather_impl(table, idx)[:, :width]


def _lrelu(v, s):
    return jnp.where(v >= 0, v, s * v)


def _head_kernel(h_ref, w1_ref, b1_ref, w2_ref, b2_ref, w3_ref, b3_ref, o_ref):
    h = h_ref[...]
    a = jnp.maximum(h @ w1_ref[...].T + b1_ref[...], 0.0)
    b = a @ w2_ref[...].T + b2_ref[...]
    o_ref[...] = b @ w3_ref[...].T + b3_ref[...]


def _head(h, params):
    n = h.shape[0]
    blk = 4096
    w1, b1 = params["h1_W"], params["h1_b"]
    w2, b2 = params["h2_W"], params["h2_b"]
    w3, b3 = params["lin_W"], params["lin_b"]
    out = pl.pallas_call(
        _head_kernel,
        grid=(n // blk,),
        in_specs=[
            pl.BlockSpec((blk, h.shape[1]), lambda i: (i, 0)),
            pl.BlockSpec(w1.shape, lambda i: (0, 0)),
            pl.BlockSpec(b1.shape, lambda i: (0,)),
            pl.BlockSpec(w2.shape, lambda i: (0, 0)),
            pl.BlockSpec(b2.shape, lambda i: (0,)),
            pl.BlockSpec(w3.shape, lambda i: (0, 0)),
            pl.BlockSpec(b3.shape, lambda i: (0,)),
        ],
        out_specs=pl.BlockSpec((blk, w3.shape[0]), lambda i: (i, 0)),
        out_shape=jax.ShapeDtypeStruct((n, w3.shape[0]), h.dtype),
    )(h, w1, b1, w2, b2, w3, b3)
    return out


_I32_MAX = jnp.iinfo(jnp.int32).max


def _aug_q(p):
    n2 = jnp.sum(p * p, axis=1, keepdims=True)
    one = jnp.ones_like(n2)
    zero = jnp.zeros((p.shape[0], 3), p.dtype)
    return jnp.concatenate([p, n2, one, zero], axis=1)


def _aug_s(p):
    n2 = jnp.sum(p * p, axis=1, keepdims=True)
    one = jnp.ones_like(n2)
    zero = jnp.zeros((p.shape[0], 3), p.dtype)
    return jnp.concatenate([-2.0 * p, one, n2, zero], axis=1)


_KNN_STRIP = 512


def _knn16_body(q_ref, s_ref, o_ref):
    tq = q_ref.shape[0]
    s_cols = s_ref.shape[0]
    strip = min(_KNN_STRIP, s_cols)
    q = q_ref[...]
    ms = tuple(jnp.full((tq, 128), _I32_MAX, jnp.int32) for _ in range(4))
    for sb in range(s_cols // strip):
        d = jax.lax.dot_general(
            q, s_ref[sb * strip:(sb + 1) * strip, :],
            (((1,), (1,)), ((), ())),
            preferred_element_type=jnp.float32,
            precision=jax.lax.Precision.HIGHEST)
        for jj in range(strip // 128):
            j = sb * (strip // 128) + jj
            m1, m2, m3, m4 = ms
            x = jnp.maximum(d[:, jj * 128:(jj + 1) * 128], 0.0)
            x = jax.lax.bitcast_convert_type(x, jnp.int32)
            x = (x & ~0xFF) | j
            t = jnp.minimum(m1, x); x = jnp.maximum(m1, x); m1 = t
            t = jnp.minimum(m2, x); x = jnp.maximum(m2, x); m2 = t
            t = jnp.minimum(m3, x); x = jnp.maximum(m3, x); m3 = t
            m4 = jnp.minimum(m4, x)
            ms = (m1, m2, m3, m4)

    cand = jnp.concatenate(ms, axis=1)
    iota = jax.lax.broadcasted_iota(jnp.int32, (tq, 512), 1)
    for kk in range(16):
        mn = jnp.min(cand, axis=1, keepdims=True)
        pos = jnp.min(jnp.where(cand == mn, iota, _I32_MAX), axis=1, keepdims=True)
        col = ((mn & 0xFF) << 7) | (pos & 127)
        o_ref[:, kk:kk + 1] = col
        cand = jnp.where(iota == pos, _I32_MAX, cand)


def _knn_idx(q, s, k, tq=None):
    assert k == 16
    qn, sn = q.shape[0], s.shape[0]
    if tq is None:
        tq = min(qn, 128)
    qa, sa = _aug_q(q), _aug_s(s)
    out = pl.pallas_call(
        _knn16_body,
        grid=(qn // tq,),
        in_specs=[
            pl.BlockSpec((tq, 8), lambda i: (i, 0)),
            pl.BlockSpec((sn, 8), lambda i: (0, 0)),
        ],
        out_specs=pl.BlockSpec((tq, 16), lambda i: (i, 0)),
        out_shape=jax.ShapeDtypeStruct((qn, 16), jnp.int32),
    )(qa, sa)
    return out


def _nn1_body(q_ref, s_ref, o_ref):
    tq = q_ref.shape[0]
    s_cols = s_ref.shape[0]
    strip = min(_KNN_STRIP, s_cols)
    q = q_ref[...]
    mv = jnp.full((tq, 128), jnp.inf, jnp.float32)
    mi = jnp.zeros((tq, 128), jnp.int32)
    for sb in range(s_cols // strip):
        d = jax.lax.dot_general(
            q, s_ref[sb * strip:(sb + 1) * strip, :],
            (((1,), (1,)), ((), ())),
            preferred_element_type=jnp.float32,
            precision=jax.lax.Precision.HIGHEST)
        for jj in range(strip // 128):
            j = sb * (strip // 128) + jj
            x = d[:, jj * 128:(jj + 1) * 128]
            upd = x < mv
            mv = jnp.where(upd, x, mv)
            mi = jnp.where(upd, j, mi)

    v = jnp.min(mv, axis=1, keepdims=True)
    lane = jax.lax.broadcasted_iota(jnp.int32, (tq, 128), 1)
    colf = (mi << 7) | lane
    col = jnp.min(jnp.where(mv == v, colf, _I32_MAX), axis=1, keepdims=True)
    o_ref[...] = col


def _nn1(q, s, tq=None):
    qn, sn = q.shape[0], s.shape[0]
    if tq is None:
        tq = min(qn, 256)
    qa, sa = _aug_q(q), _aug_s(s)
    out = pl.pallas_call(
        _nn1_body,
        grid=(qn // tq,),
        in_specs=[
            pl.BlockSpec((tq, 8), lambda i: (i, 0)),
            pl.BlockSpec((sn, 8), lambda i: (0, 0)),
        ],
        out_specs=pl.BlockSpec((tq, 1), lambda i: (i, 0)),
        out_shape=jax.ShapeDtypeStruct((qn, 1), jnp.int32),
    )(qa, sa)
    return out.reshape(qn)


def _lfa_edges(params, pfx, x_j, pos_i, pos_j, n_q, k):
    dist = pos_j - pos_i
    eu = jnp.sum(jnp.abs(dist), axis=1, keepdims=True)
    rel = jnp.concatenate([pos_i, pos_j, dist, eu], axis=1)
    lse = rel @ params[pfx + "e_W"].T + params[pfx + "e_b"]
    out1 = jnp.concatenate([x_j, lse], axis=1)
    att = jax.nn.softmax(out1 @ params[pfx + "a_W"].T + params[pfx + "a_b"], axis=-1)
    msg = att * out1
    return msg.reshape(n_q, k, msg.shape[1]).sum(axis=1)


def _block(params, pfx, x, pos, decimation, k):
    n = x.shape[0]
    idx = jnp.arange(0, n, decimation)
    n_q = idx.shape[0]
    q_pos = pos[idx]
    nbrs = _knn_idx(q_pos, pos, k)
    col = nbrs.reshape(-1)

    h0 = _lrelu(x @ params[pfx + "_m1_W"].T + params[pfx + "_m1_b"], 0.2)
    c1 = h0.shape[1]
    g1 = _sc_gather(jnp.concatenate([h0, pos], axis=1), col, c1 + 3)
    x_j1, pos_j = g1[:, :c1], g1[:, c1:]
    pos_i = jnp.repeat(pos[:n_q], k, axis=0)
    h1 = _lfa_edges(params, pfx + "_l1_", x_j1, pos_i, pos_j, n_q, k)
    x_j2 = _sc_gather(h1, jnp.minimum(col, n_q - 1), h1.shape[1])
    x_j2 = jnp.where((col < n_q)[:, None], x_j2, 0.0)
    h2 = _lfa_edges(params, pfx + "_l2_", x_j2, pos_i, pos_j, n_q, k)
    h2_idx = jnp.where((idx < n_q)[:, None], h2[jnp.minimum(idx, n_q - 1)], 0.0)
    m2 = _lrelu(h2_idx @ params[pfx + "_m2_W"].T + params[pfx + "_m2_b"], 0.2)
    sc = _lrelu(x[idx] @ params[pfx + "_sc_W"].T + params[pfx + "_sc_b"], 0.2)
    out = _lrelu(m2 + sc, 0.01)
    return out, q_pos


def _fp(params, pfx, xh, pos, pos_skip, x_skip):
    nn = _nn1(pos_skip, pos)
    xi = _sc_gather(xh, nn, xh.shape[1])
    if x_skip is not None:
        xi = jnp.concatenate([xi, x_skip], axis=1)
    return xi @ params[pfx + "_W"].T + params[pfx + "_b"]


@jax.jit
def _forward(x, pos, params):
    x0, p0 = x, pos
    x1, p1 = _block(params, "b1", x0, p0, 4, 16)
    x2, p2 = _block(params, "b2", x1, p1, 4, 16)
    x3, p3 = _block(params, "b3", x2, p2, 4, 16)
    x4, p4 = _block(params, "b4", x3, p3, 4, 16)
    h = x4 @ params["mlp1_W"].T + params["mlp1_b"]
    h = _fp(params, "fp4", h, p4, p3, x3)
    h = _fp(params, "fp3", h, p3, p2, x2)
    h = _fp(params, "fp2", h, p2, p1, x1)
    h = _fp(params, "fp1", h, p1, p0, x0)
    return _head(h, params)


def kernel(x, pos, batch, params):
    return _forward(x, pos, params)
